# baseline (speedup 1.0000x reference)
import functools
import jax
import jax.numpy as jnp
from jax import lax
from jax.experimental import pallas as pl
from jax.experimental.pallas import tpu as pltpu
from jax.experimental.pallas import tpu_sc as plsc

_N = 10000
_E = 320000
_D = 128
_H = 8
_F = 8
_C = 3
_HF = _H * _F
_MB = 128
_CH = 512
_NCHUNK = _E // _CH
_NW = 32
_ROWS_PER_TILE = 625
f32 = jnp.float32


def _vgather(x, idx):
    return lax.gather(
        x, idx[:, None],
        dimension_numbers=lax.GatherDimensionNumbers(
            offset_dims=(), collapsed_slice_dims=(0,), start_index_map=(0,)),
        slice_sizes=(1,),
        mode=lax.GatherScatterMode.PROMISE_IN_BOUNDS)


def _stage_a_body(x_ref, wlo_ref, whi_ref, wes_ref, wed_ref,
                  hw_ref, es_ref, ed_ref):
    lo = jnp.dot(x_ref[...], wlo_ref[...], preferred_element_type=f32)
    hi = jnp.dot(x_ref[...], whi_ref[...], preferred_element_type=f32)
    lb = lax.bitcast_convert_type(lo, jnp.int32)
    hb = lax.bitcast_convert_type(hi, jnp.int32)
    rl = lb + 0x7FFF + ((lb >> 16) & 1)
    rh = hb + 0x7FFF + ((hb >> 16) & 1)
    hw_ref[...] = lax.shift_right_logical(rl, 16) | (rh & (-65536))
    es_ref[...] = jnp.dot(x_ref[...], wes_ref[...], preferred_element_type=f32)
    ed_ref[...] = jnp.dot(x_ref[...], wed_ref[...], preferred_element_type=f32)


def _stage_a(x, wlo, whi, wes, wed):
    return pl.pallas_call(
        _stage_a_body,
        out_shape=(
            jax.ShapeDtypeStruct((_N, _HF // 2), jnp.int32),
            jax.ShapeDtypeStruct((_N, _H), f32),
            jax.ShapeDtypeStruct((_N, _H), f32),
        ),
    )(x, wlo, whi, wes, wed)


_MAXK = 79


def _worker_range(w):
    start = 78 * w + jnp.minimum(w, 4)
    nk = 78 + (w < 4).astype(jnp.int32)
    return start, nk


def _prefetch_idx(src_hbm, dst_hbm, w, sidx, didx):
    start, nk = _worker_range(w)

    @pl.when(w < 4)
    def _():
        pltpu.sync_copy(src_hbm.at[pl.ds(start, 79)], sidx)
        pltpu.sync_copy(dst_hbm.at[pl.ds(start, 79)], didx)

    @pl.when(w >= 4)
    def _():
        pltpu.sync_copy(src_hbm.at[pl.ds(start, 78)], sidx.at[pl.ds(0, 78)])
        pltpu.sync_copy(dst_hbm.at[pl.ds(start, 78)], didx.at[pl.ds(0, 78)])

    return nk


def _sc1_body(src_hbm, dst_hbm, hbf_hbm, es_hbm, ed_hbm,
              msg_out, den_out,
              sidx, didx, esb, edb, hbb, msgb, exs, macc, dacc,
              sg0, sg1, sg2, ss0, ss1):
    c = lax.axis_index("c")
    s = lax.axis_index("s")
    w = s * 2 + c
    iot = lax.broadcasted_iota(jnp.int32, (16,), 0)
    rowoff = iot >> 3
    colv = iot & 7
    zero16 = jnp.zeros((16,), f32)
    sgs = (sg0, sg1, sg2)
    sss = (ss0, ss1)

    @pl.loop(0, _MB)
    def _zero(r):
        for j in range(4):
            msgb[0, r, pl.ds(16 * j, 16)] = zero16

    @pl.loop(0, 64)
    def _zero2(r):
        plsc.store_scatter(exs.at[0], [2 * r + rowoff, colv], zero16)

    base = s * _ROWS_PER_TILE
    for k in range(5):
        pltpu.sync_copy(msgb.at[0, pl.ds(0, 125)],
                        macc.at[pl.ds(base + 125 * k, 125)])

    @pl.when(s < 15)
    def _():
        for k in range(5):
            pltpu.sync_copy(exs.at[0], dacc.at[pl.ds(s * 640 + 128 * k, _MB)])

    @pl.when(s == 15)
    def _():
        for k in range(3):
            pltpu.sync_copy(exs.at[0], dacc.at[pl.ds(9600 + 128 * k, _MB)])
        pltpu.sync_copy(exs.at[0, pl.ds(0, 16)], dacc.at[pl.ds(9984, 16)])

    plsc.subcore_barrier()

    nk = _prefetch_idx(src_hbm, dst_hbm, w, sidx, didx)

    def issue_gathers(b, k):
        pltpu.async_copy(hbf_hbm.at[sidx.at[k]], hbb.at[b], sgs[b])
        pltpu.async_copy(es_hbm.at[sidx.at[k]], esb.at[b], sgs[b])
        pltpu.async_copy(ed_hbm.at[didx.at[k]], edb.at[b], sgs[b])

    def wait_gathers(b):
        pltpu.make_async_copy(hbf_hbm.at[sidx.at[0]], hbb.at[b], sgs[b]).wait()
        pltpu.make_async_copy(es_hbm.at[sidx.at[0]], esb.at[b], sgs[b]).wait()
        pltpu.make_async_copy(ed_hbm.at[didx.at[0]], edb.at[b], sgs[b]).wait()

    def issue_scatter(b, k):
        pltpu.async_copy(msgb.at[b], macc.at[didx.at[k]], sss[b], add=True)
        pltpu.async_copy(exs.at[b], dacc.at[didx.at[k]], sss[b], add=True)

    def drain_scatter(b):
        pltpu.make_async_copy(msgb.at[b], macc.at[didx.at[0]], sss[b]).wait()
        pltpu.make_async_copy(exs.at[b], dacc.at[didx.at[0]], sss[b]).wait()

    idxa = [[e01 * 8 + 4 * q + rowoff for q in range(2)] for e01 in range(2)]
    idxb = [[e01 * 8 + 4 * q + 2 + rowoff for q in range(2)]
            for e01 in range(2)]

    def compute(gb, sb):
        @pl.loop(0, 64, unroll=4)
        def _pair(p):
            r2 = 2 * p + rowoff
            esv = plsc.load_gather(esb.at[gb], [r2, colv])
            edv = plsc.load_gather(edb.at[gb], [r2, colv])
            t = esv + edv
            exv = jnp.exp(jnp.maximum(t, 0.2 * t))
            plsc.store_scatter(exs.at[sb], [r2, colv], exv)
            for e01 in range(2):
                cc = 2 * p + e01
                for q in range(2):
                    wv = hbb[gb, cc, pl.ds(16 * q, 16)]
                    ha = plsc.bitcast(wv << 16, f32)
                    hbv = plsc.bitcast(wv & (-65536), f32)
                    msgb[sb, cc, pl.ds(32 * q, 16)] = (
                        ha * _vgather(exv, idxa[e01][q]))
                    msgb[sb, cc, pl.ds(32 * q + 16, 16)] = (
                        hbv * _vgather(exv, idxb[e01][q]))

    issue_gathers(0, 0)

    @pl.when(1 < nk)
    def _():
        issue_gathers(1, 1)

    @pl.loop(0, (_MAXK + 5) // 6)
    def _outer(it):
        for b in range(6):
            k = 6 * it + b
            gb = b % 3
            sb = b % 2

            @pl.when(k < nk)
            def _():
                wait_gathers(gb)

                @pl.when(k + 2 < nk)
                def _():
                    issue_gathers((b + 2) % 3, k + 2)

                @pl.when(k >= 2)
                def _():
                    drain_scatter(sb)

                compute(gb, sb)
                issue_scatter(sb, k)

    drain_scatter(0)
    drain_scatter(1)

    plsc.subcore_barrier()
    pltpu.sync_copy(macc.at[pl.ds(base, _ROWS_PER_TILE)],
                    msg_out.at[c, pl.ds(base, _ROWS_PER_TILE)])

    @pl.when(s < 15)
    def _():
        pltpu.sync_copy(dacc.at[pl.ds(s * 640, 640)],
                        den_out.at[c, pl.ds(s * 640, 640)])

    @pl.when(s == 15)
    def _():
        pltpu.sync_copy(dacc.at[pl.ds(9600, 400)],
                        den_out.at[c, pl.ds(9600, 400)])


def _sc1(src2d, dst2d, hbf, es, ed):
    fn = pl.kernel(
        _sc1_body,
        out_type=(
            jax.ShapeDtypeStruct((2, _N, _HF), f32),
            jax.ShapeDtypeStruct((2, _N, _H), f32),
        ),
        mesh=plsc.VectorSubcoreMesh(core_axis_name="c", subcore_axis_name="s"),
        compiler_params=pltpu.CompilerParams(use_tc_tiling_on_sc=False,
                                             needs_layout_passes=False),
        scratch_types=[
            pltpu.VMEM((_MAXK, _MB), jnp.int32),
            pltpu.VMEM((_MAXK, _MB), jnp.int32),
            pltpu.VMEM((3, _MB, _H), f32),
            pltpu.VMEM((3, _MB, _H), f32),
            pltpu.VMEM((3, _MB, _HF // 2), jnp.int32),
            pltpu.VMEM((2, _MB, _HF), f32),
            pltpu.VMEM((2, _MB, _H), f32),
            pltpu.VMEM_SHARED((_N, _HF), f32),
            pltpu.VMEM_SHARED((_N, _H), f32),
            pltpu.SemaphoreType.DMA,
            pltpu.SemaphoreType.DMA,
            pltpu.SemaphoreType.DMA,
            pltpu.SemaphoreType.DMA,
            pltpu.SemaphoreType.DMA,
        ],
    )
    return fn(src2d, dst2d, hbf, es, ed)


def _stage_b_body(msgp_ref, denp_ref, r8_ref, wsrc_ref, wdst_ref,
                  stab_ref, dtab_ref):
    msg = msgp_ref[0] + msgp_ref[1]
    den = denp_ref[0] + denp_ref[1]
    den64 = jnp.dot(den, r8_ref[...], preferred_element_type=f32) + 1e-16
    x2 = msg / den64
    x2 = jnp.where(x2 > 0, x2, jnp.exp(x2) - 1.0)
    colx = lax.broadcasted_iota(jnp.int32, (_N, _H), 1)
    stab_ref[...] = (jnp.dot(x2, wsrc_ref[...], preferred_element_type=f32)
                     + jnp.where(colx == 4, 1.0, 0.0))
    dtab_ref[...] = jnp.dot(x2, wdst_ref[...], preferred_element_type=f32)


def _stage_b(msgp, denp, r8, wsrc, wdst):
    return pl.pallas_call(
        _stage_b_body,
        out_shape=(
            jax.ShapeDtypeStruct((_N, _H), f32),
            jax.ShapeDtypeStruct((_N, _H), f32),
        ),
    )(msgp, denp, r8, wsrc, wdst)


def _sc2_body(src_hbm, dst_hbm, stab_hbm, dtab_hbm, acc_out,
              sidx, didx, sB, dB, mb, acc2,
              sg0, sg1, sg2, ss0, ss1):
    c = lax.axis_index("c")
    s = lax.axis_index("s")
    w = s * 2 + c
    iot = lax.broadcasted_iota(jnp.int32, (16,), 0)
    rowoff = iot >> 3
    colv = iot & 7
    zero16 = jnp.zeros((16,), f32)
    sgs = (sg0, sg1, sg2)
    sss = (ss0, ss1)

    @pl.loop(0, 64)
    def _zero(r):
        plsc.store_scatter(mb.at[0], [2 * r + rowoff, colv], zero16)

    @pl.when(s < 15)
    def _():
        for k in range(5):
            pltpu.sync_copy(mb.at[0], acc2.at[pl.ds(s * 640 + 128 * k, _MB)])

    @pl.when(s == 15)
    def _():
        for k in range(3):
            pltpu.sync_copy(mb.at[0], acc2.at[pl.ds(9600 + 128 * k, _MB)])
        pltpu.sync_copy(mb.at[0, pl.ds(0, 16)], acc2.at[pl.ds(9984, 16)])

    plsc.subcore_barrier()

    nk = _prefetch_idx(src_hbm, dst_hbm, w, sidx, didx)

    def issue_gathers(b, k):
        pltpu.async_copy(stab_hbm.at[sidx.at[k]], sB.at[b], sgs[b])
        pltpu.async_copy(dtab_hbm.at[didx.at[k]], dB.at[b], sgs[b])

    def wait_gathers(b):
        pltpu.make_async_copy(stab_hbm.at[sidx.at[0]], sB.at[b], sgs[b]).wait()
        pltpu.make_async_copy(dtab_hbm.at[didx.at[0]], dB.at[b], sgs[b]).wait()

    bidx = rowoff * 8

    def compute(gb, sb):
        @pl.loop(0, 64, unroll=8)
        def _pair(v):
            r2 = 2 * v + rowoff
            sv = plsc.load_gather(sB.at[gb], [r2, colv])
            dv = plsc.load_gather(dB.at[gb], [r2, colv])
            t = sv + dv
            ex = jnp.exp(jnp.maximum(t, 0.2 * t))
            exe = _vgather(ex, bidx)
            plsc.store_scatter(mb.at[sb], [r2, colv], exe * sv)

    issue_gathers(0, 0)

    @pl.when(1 < nk)
    def _():
        issue_gathers(1, 1)

    @pl.loop(0, (_MAXK + 5) // 6)
    def _outer(it):
        for b in range(6):
            k = 6 * it + b
            gb = b % 3
            sb = b % 2

            @pl.when(k < nk)
            def _():
                wait_gathers(gb)

                @pl.when(k + 2 < nk)
                def _():
                    issue_gathers((b + 2) % 3, k + 2)

                @pl.when(k >= 2)
                def _():
                    pltpu.make_async_copy(mb.at[sb], acc2.at[didx.at[0]],
                                          sss[sb]).wait()

                compute(gb, sb)
                pltpu.async_copy(mb.at[sb], acc2.at[didx.at[k]], sss[sb],
                                 add=True)

    pltpu.make_async_copy(mb.at[0], acc2.at[didx.at[0]], sss[0]).wait()
    pltpu.make_async_copy(mb.at[1], acc2.at[didx.at[0]], sss[1]).wait()

    plsc.subcore_barrier()

    @pl.when(s < 15)
    def _():
        pltpu.sync_copy(acc2.at[pl.ds(s * 640, 640)],
                        acc_out.at[c, pl.ds(s * 640, 640)])

    @pl.when(s == 15)
    def _():
        pltpu.sync_copy(acc2.at[pl.ds(9600, 400)],
                        acc_out.at[c, pl.ds(9600, 400)])


def _sc2(src2d, dst2d, stab, dtab):
    fn = pl.kernel(
        _sc2_body,
        out_type=jax.ShapeDtypeStruct((2, _N, _H), f32),
        mesh=plsc.VectorSubcoreMesh(core_axis_name="c", subcore_axis_name="s"),
        compiler_params=pltpu.CompilerParams(use_tc_tiling_on_sc=False,
                                             needs_layout_passes=False),
        scratch_types=[
            pltpu.VMEM((_MAXK, _MB), jnp.int32),
            pltpu.VMEM((_MAXK, _MB), jnp.int32),
            pltpu.VMEM((3, _MB, _H), f32),
            pltpu.VMEM((3, _MB, _H), f32),
            pltpu.VMEM((2, _MB, _H), f32),
            pltpu.VMEM_SHARED((_N, _H), f32),
            pltpu.SemaphoreType.DMA,
            pltpu.SemaphoreType.DMA,
            pltpu.SemaphoreType.DMA,
            pltpu.SemaphoreType.DMA,
            pltpu.SemaphoreType.DMA,
        ],
    )
    return fn(src2d, dst2d, stab, dtab)


def _stage_c_body(accp_ref, out_ref):
    a = accp_ref[0] + accp_ref[1]
    colx = lax.broadcasted_iota(jnp.int32, (_N, _H), 1)
    den = jnp.sum(jnp.where(colx == 4, a, 0.0), axis=1, keepdims=True)
    x = a / (den + 1e-16)
    valid = (colx >= 1) & (colx <= 3)
    z = jnp.where(valid, x, -1e30)
    m = jnp.max(z, axis=1, keepdims=True)
    ez = jnp.where(valid, jnp.exp(z - m), 0.0)
    out_ref[...] = ez / jnp.sum(ez, axis=1, keepdims=True)


def _stage_c(accp):
    return pl.pallas_call(
        _stage_c_body,
        out_shape=jax.ShapeDtypeStruct((_N, _H), f32),
    )(accp)


@jax.jit
def kernel(node_embeddings, edge_index, W1, a1_src, a1_dst, W2, a2_src, a2_dst):
    w1r = W1.reshape(_D, _HF)
    hf_ids = jnp.arange(_HF, dtype=jnp.int32)
    asrc = jnp.zeros((_HF, _H), f32).at[hf_ids, hf_ids // _F].set(
        a1_src.reshape(-1))
    adst = jnp.zeros((_HF, _H), f32).at[hf_ids, hf_ids // _F].set(
        a1_dst.reshape(-1))
    m16 = jnp.arange(16, dtype=jnp.int32)
    cols_a = jnp.concatenate([m16, 32 + m16])
    wlo = w1r[:, cols_a]
    whi = w1r[:, cols_a + 16]
    wes = w1r @ asrc
    wed = w1r @ adst
    r8 = jnp.zeros((_H, _HF), f32).at[hf_ids // _F, hf_ids].set(1.0)
    w2r = W2.reshape(_HF, _C)
    wsrc = jnp.zeros((_HF, _H), f32)
    wsrc = wsrc.at[:, 0].set(w2r @ a2_src.reshape(_C))
    wsrc = wsrc.at[:, 1:4].set(w2r)
    wdst = jnp.zeros((_HF, _H), f32).at[:, 0].set(w2r @ a2_dst.reshape(_C))

    src2d = edge_index[0].reshape(_E // _MB, _MB)
    dst2d = edge_index[1].reshape(_E // _MB, _MB)

    hw, es, ed = _stage_a(node_embeddings, wlo, whi, wes, wed)
    msgp, denp = _sc1(src2d, dst2d, hw, es, ed)
    stab, dtab = _stage_b(msgp, denp, r8, wsrc, wdst)
    accp = _sc2(src2d, dst2d, stab, dtab)
    out8 = _stage_c(accp)
    return out8[:, 1:4]


# speedup vs baseline: 1.0274x; 1.0274x over previous
import functools
import jax
import jax.numpy as jnp
from jax import lax
from jax.experimental import pallas as pl
from jax.experimental.pallas import tpu as pltpu
from jax.experimental.pallas import tpu_sc as plsc

_N = 10000
_E = 320000
_D = 128
_H = 8
_F = 8
_C = 3
_HF = _H * _F
_MB = 128
_CH = 512
_NCHUNK = _E // _CH
_NW = 32
_ROWS_PER_TILE = 625
f32 = jnp.float32


def _vgather(x, idx):
    return lax.gather(
        x, idx[:, None],
        dimension_numbers=lax.GatherDimensionNumbers(
            offset_dims=(), collapsed_slice_dims=(0,), start_index_map=(0,)),
        slice_sizes=(1,),
        mode=lax.GatherScatterMode.PROMISE_IN_BOUNDS)


def _stage_a_body(x_ref, wlo_ref, whi_ref, wes_ref, wed_ref,
                  hw_ref, es_ref, ed_ref):
    lo = jnp.dot(x_ref[...], wlo_ref[...], preferred_element_type=f32)
    hi = jnp.dot(x_ref[...], whi_ref[...], preferred_element_type=f32)
    lb = lax.bitcast_convert_type(lo, jnp.int32)
    hb = lax.bitcast_convert_type(hi, jnp.int32)
    rl = lb + 0x7FFF + ((lb >> 16) & 1)
    rh = hb + 0x7FFF + ((hb >> 16) & 1)
    hw_ref[...] = lax.shift_right_logical(rl, 16) | (rh & (-65536))
    es_ref[...] = jnp.dot(x_ref[...], wes_ref[...], preferred_element_type=f32)
    ed_ref[...] = jnp.dot(x_ref[...], wed_ref[...], preferred_element_type=f32)


def _stage_a(x, wlo, whi, wes, wed):
    return pl.pallas_call(
        _stage_a_body,
        out_shape=(
            jax.ShapeDtypeStruct((_N, _HF // 2), jnp.int32),
            jax.ShapeDtypeStruct((_N, _H), f32),
            jax.ShapeDtypeStruct((_N, _H), f32),
        ),
    )(x, wlo, whi, wes, wed)


_MAXK = 79


def _worker_range(w):
    start = 78 * w + jnp.minimum(w, 4)
    nk = 78 + (w < 4).astype(jnp.int32)
    return start, nk


def _prefetch_idx_start(src_hbm, dst_hbm, w, sidx, didx, sem):
    start, nk = _worker_range(w)

    @pl.when(w < 4)
    def _():
        pltpu.async_copy(src_hbm.at[pl.ds(start, 79)], sidx, sem)
        pltpu.async_copy(dst_hbm.at[pl.ds(start, 79)], didx, sem)

    @pl.when(w >= 4)
    def _():
        pltpu.async_copy(src_hbm.at[pl.ds(start, 78)],
                         sidx.at[pl.ds(0, 78)], sem)
        pltpu.async_copy(dst_hbm.at[pl.ds(start, 78)],
                         didx.at[pl.ds(0, 78)], sem)

    return nk


def _prefetch_idx_wait(src_hbm, dst_hbm, w, sidx, didx, sem):
    start, _ = _worker_range(w)

    @pl.when(w < 4)
    def _():
        pltpu.make_async_copy(src_hbm.at[pl.ds(start, 79)], sidx, sem).wait()
        pltpu.make_async_copy(dst_hbm.at[pl.ds(start, 79)], didx, sem).wait()

    @pl.when(w >= 4)
    def _():
        pltpu.make_async_copy(src_hbm.at[pl.ds(start, 78)],
                              sidx.at[pl.ds(0, 78)], sem).wait()
        pltpu.make_async_copy(dst_hbm.at[pl.ds(start, 78)],
                              didx.at[pl.ds(0, 78)], sem).wait()


def _sc1_body(src_hbm, dst_hbm, hbf_hbm, es_hbm, ed_hbm,
              msg_out, den_out,
              sidx, didx, esb, edb, hbb, msgb, exs, macc, dacc,
              sg0, sg1, sg2, ss0, ss1):
    c = lax.axis_index("c")
    s = lax.axis_index("s")
    w = s * 2 + c
    iot = lax.broadcasted_iota(jnp.int32, (16,), 0)
    rowoff = iot >> 3
    colv = iot & 7
    zero16 = jnp.zeros((16,), f32)
    sgs = (sg0, sg1, sg2)
    sss = (ss0, ss1)
    nk = _prefetch_idx_start(src_hbm, dst_hbm, w, sidx, didx, ss0)

    @pl.loop(0, _MB)
    def _zero(r):
        for j in range(4):
            msgb[0, r, pl.ds(16 * j, 16)] = zero16

    @pl.loop(0, 64)
    def _zero2(r):
        plsc.store_scatter(exs.at[0], [2 * r + rowoff, colv], zero16)

    base = s * _ROWS_PER_TILE
    for k in range(5):
        pltpu.sync_copy(msgb.at[0, pl.ds(0, 125)],
                        macc.at[pl.ds(base + 125 * k, 125)])

    @pl.when(s < 15)
    def _():
        for k in range(5):
            pltpu.sync_copy(exs.at[0], dacc.at[pl.ds(s * 640 + 128 * k, _MB)])

    @pl.when(s == 15)
    def _():
        for k in range(3):
            pltpu.sync_copy(exs.at[0], dacc.at[pl.ds(9600 + 128 * k, _MB)])
        pltpu.sync_copy(exs.at[0, pl.ds(0, 16)], dacc.at[pl.ds(9984, 16)])

    _prefetch_idx_wait(src_hbm, dst_hbm, w, sidx, didx, ss0)
    plsc.subcore_barrier()

    def issue_gathers(b, k):
        pltpu.async_copy(hbf_hbm.at[sidx.at[k]], hbb.at[b], sgs[b])
        pltpu.async_copy(es_hbm.at[sidx.at[k]], esb.at[b], sgs[b])
        pltpu.async_copy(ed_hbm.at[didx.at[k]], edb.at[b], sgs[b])

    def wait_gathers(b):
        pltpu.make_async_copy(hbf_hbm.at[sidx.at[0]], hbb.at[b], sgs[b]).wait()
        pltpu.make_async_copy(es_hbm.at[sidx.at[0]], esb.at[b], sgs[b]).wait()
        pltpu.make_async_copy(ed_hbm.at[didx.at[0]], edb.at[b], sgs[b]).wait()

    def issue_scatter(b, k):
        pltpu.async_copy(msgb.at[b], macc.at[didx.at[k]], sss[b], add=True)
        pltpu.async_copy(exs.at[b], dacc.at[didx.at[k]], sss[b], add=True)

    def drain_scatter(b):
        pltpu.make_async_copy(msgb.at[b], macc.at[didx.at[0]], sss[b]).wait()
        pltpu.make_async_copy(exs.at[b], dacc.at[didx.at[0]], sss[b]).wait()

    idxa = [[e01 * 8 + 4 * q + rowoff for q in range(2)] for e01 in range(2)]
    idxb = [[e01 * 8 + 4 * q + 2 + rowoff for q in range(2)]
            for e01 in range(2)]

    def compute(gb, sb):
        @pl.loop(0, 64, unroll=2)
        def _pair(p):
            r2 = 2 * p + rowoff
            esv = plsc.load_gather(esb.at[gb], [r2, colv])
            edv = plsc.load_gather(edb.at[gb], [r2, colv])
            t = esv + edv
            exv = jnp.exp(jnp.maximum(t, 0.2 * t))
            plsc.store_scatter(exs.at[sb], [r2, colv], exv)
            for e01 in range(2):
                cc = 2 * p + e01
                for q in range(2):
                    wv = hbb[gb, cc, pl.ds(16 * q, 16)]
                    ha = plsc.bitcast(wv << 16, f32)
                    hbv = plsc.bitcast(wv & (-65536), f32)
                    msgb[sb, cc, pl.ds(32 * q, 16)] = (
                        ha * _vgather(exv, idxa[e01][q]))
                    msgb[sb, cc, pl.ds(32 * q + 16, 16)] = (
                        hbv * _vgather(exv, idxb[e01][q]))

    issue_gathers(0, 0)

    @pl.when(1 < nk)
    def _():
        issue_gathers(1, 1)

    @pl.loop(0, (_MAXK + 5) // 6)
    def _outer(it):
        for b in range(6):
            k = 6 * it + b
            gb = b % 3
            sb = b % 2

            @pl.when(k < nk)
            def _():
                wait_gathers(gb)

                @pl.when(k + 2 < nk)
                def _():
                    issue_gathers((b + 2) % 3, k + 2)

                @pl.when(k >= 2)
                def _():
                    drain_scatter(sb)

                compute(gb, sb)
                issue_scatter(sb, k)

    drain_scatter(0)
    drain_scatter(1)

    plsc.subcore_barrier()
    pltpu.sync_copy(macc.at[pl.ds(base, _ROWS_PER_TILE)],
                    msg_out.at[c, pl.ds(base, _ROWS_PER_TILE)])

    @pl.when(s < 15)
    def _():
        pltpu.sync_copy(dacc.at[pl.ds(s * 640, 640)],
                        den_out.at[c, pl.ds(s * 640, 640)])

    @pl.when(s == 15)
    def _():
        pltpu.sync_copy(dacc.at[pl.ds(9600, 400)],
                        den_out.at[c, pl.ds(9600, 400)])


def _sc1(src2d, dst2d, hbf, es, ed):
    fn = pl.kernel(
        _sc1_body,
        out_type=(
            jax.ShapeDtypeStruct((2, _N, _HF), f32),
            jax.ShapeDtypeStruct((2, _N, _H), f32),
        ),
        mesh=plsc.VectorSubcoreMesh(core_axis_name="c", subcore_axis_name="s"),
        compiler_params=pltpu.CompilerParams(use_tc_tiling_on_sc=False,
                                             needs_layout_passes=False),
        scratch_types=[
            pltpu.VMEM((_MAXK, _MB), jnp.int32),
            pltpu.VMEM((_MAXK, _MB), jnp.int32),
            pltpu.VMEM((3, _MB, _H), f32),
            pltpu.VMEM((3, _MB, _H), f32),
            pltpu.VMEM((3, _MB, _HF // 2), jnp.int32),
            pltpu.VMEM((2, _MB, _HF), f32),
            pltpu.VMEM((2, _MB, _H), f32),
            pltpu.VMEM_SHARED((_N, _HF), f32),
            pltpu.VMEM_SHARED((_N, _H), f32),
            pltpu.SemaphoreType.DMA,
            pltpu.SemaphoreType.DMA,
            pltpu.SemaphoreType.DMA,
            pltpu.SemaphoreType.DMA,
            pltpu.SemaphoreType.DMA,
        ],
    )
    return fn(src2d, dst2d, hbf, es, ed)


def _stage_b_body(msgp_ref, denp_ref, r8_ref, wsrc_ref, wdst_ref,
                  stab_ref, dtab_ref):
    msg = msgp_ref[0] + msgp_ref[1]
    den = denp_ref[0] + denp_ref[1]
    den64 = jnp.dot(den, r8_ref[...], preferred_element_type=f32) + 1e-16
    x2 = msg / den64
    x2 = jnp.where(x2 > 0, x2, jnp.exp(x2) - 1.0)
    colx = lax.broadcasted_iota(jnp.int32, (_N, _H), 1)
    stab_ref[...] = (jnp.dot(x2, wsrc_ref[...], preferred_element_type=f32)
                     + jnp.where(colx == 4, 1.0, 0.0))
    dtab_ref[...] = jnp.dot(x2, wdst_ref[...], preferred_element_type=f32)


def _stage_b(msgp, denp, r8, wsrc, wdst):
    return pl.pallas_call(
        _stage_b_body,
        out_shape=(
            jax.ShapeDtypeStruct((_N, _H), f32),
            jax.ShapeDtypeStruct((_N, _H), f32),
        ),
    )(msgp, denp, r8, wsrc, wdst)


def _sc2_body(src_hbm, dst_hbm, stab_hbm, dtab_hbm, acc_out,
              sidx, didx, sB, dB, mb, acc2,
              sg0, sg1, sg2, ss0, ss1):
    c = lax.axis_index("c")
    s = lax.axis_index("s")
    w = s * 2 + c
    iot = lax.broadcasted_iota(jnp.int32, (16,), 0)
    rowoff = iot >> 3
    colv = iot & 7
    zero16 = jnp.zeros((16,), f32)
    sgs = (sg0, sg1, sg2)
    sss = (ss0, ss1)
    nk = _prefetch_idx_start(src_hbm, dst_hbm, w, sidx, didx, ss0)

    @pl.loop(0, 64)
    def _zero(r):
        plsc.store_scatter(mb.at[0], [2 * r + rowoff, colv], zero16)

    @pl.when(s < 15)
    def _():
        for k in range(5):
            pltpu.sync_copy(mb.at[0], acc2.at[pl.ds(s * 640 + 128 * k, _MB)])

    @pl.when(s == 15)
    def _():
        for k in range(3):
            pltpu.sync_copy(mb.at[0], acc2.at[pl.ds(9600 + 128 * k, _MB)])
        pltpu.sync_copy(mb.at[0, pl.ds(0, 16)], acc2.at[pl.ds(9984, 16)])

    _prefetch_idx_wait(src_hbm, dst_hbm, w, sidx, didx, ss0)
    plsc.subcore_barrier()

    def issue_gathers(b, k):
        pltpu.async_copy(stab_hbm.at[sidx.at[k]], sB.at[b], sgs[b])
        pltpu.async_copy(dtab_hbm.at[didx.at[k]], dB.at[b], sgs[b])

    def wait_gathers(b):
        pltpu.make_async_copy(stab_hbm.at[sidx.at[0]], sB.at[b], sgs[b]).wait()
        pltpu.make_async_copy(dtab_hbm.at[didx.at[0]], dB.at[b], sgs[b]).wait()

    bidx = rowoff * 8

    def compute(gb, sb):
        @pl.loop(0, 64, unroll=4)
        def _pair(v):
            r2 = 2 * v + rowoff
            sv = plsc.load_gather(sB.at[gb], [r2, colv])
            dv = plsc.load_gather(dB.at[gb], [r2, colv])
            t = sv + dv
            ex = jnp.exp(jnp.maximum(t, 0.2 * t))
            exe = _vgather(ex, bidx)
            plsc.store_scatter(mb.at[sb], [r2, colv], exe * sv)

    issue_gathers(0, 0)

    @pl.when(1 < nk)
    def _():
        issue_gathers(1, 1)

    @pl.loop(0, (_MAXK + 5) // 6)
    def _outer(it):
        for b in range(6):
            k = 6 * it + b
            gb = b % 3
            sb = b % 2

            @pl.when(k < nk)
            def _():
                wait_gathers(gb)

                @pl.when(k + 2 < nk)
                def _():
                    issue_gathers((b + 2) % 3, k + 2)

                @pl.when(k >= 2)
                def _():
                    pltpu.make_async_copy(mb.at[sb], acc2.at[didx.at[0]],
                                          sss[sb]).wait()

                compute(gb, sb)
                pltpu.async_copy(mb.at[sb], acc2.at[didx.at[k]], sss[sb],
                                 add=True)

    pltpu.make_async_copy(mb.at[0], acc2.at[didx.at[0]], sss[0]).wait()
    pltpu.make_async_copy(mb.at[1], acc2.at[didx.at[0]], sss[1]).wait()

    plsc.subcore_barrier()

    @pl.when(s < 15)
    def _():
        pltpu.sync_copy(acc2.at[pl.ds(s * 640, 640)],
                        acc_out.at[c, pl.ds(s * 640, 640)])

    @pl.when(s == 15)
    def _():
        pltpu.sync_copy(acc2.at[pl.ds(9600, 400)],
                        acc_out.at[c, pl.ds(9600, 400)])


def _sc2(src2d, dst2d, stab, dtab):
    fn = pl.kernel(
        _sc2_body,
        out_type=jax.ShapeDtypeStruct((2, _N, _H), f32),
        mesh=plsc.VectorSubcoreMesh(core_axis_name="c", subcore_axis_name="s"),
        compiler_params=pltpu.CompilerParams(use_tc_tiling_on_sc=False,
                                             needs_layout_passes=False),
        scratch_types=[
            pltpu.VMEM((_MAXK, _MB), jnp.int32),
            pltpu.VMEM((_MAXK, _MB), jnp.int32),
            pltpu.VMEM((3, _MB, _H), f32),
            pltpu.VMEM((3, _MB, _H), f32),
            pltpu.VMEM((2, _MB, _H), f32),
            pltpu.VMEM_SHARED((_N, _H), f32),
            pltpu.SemaphoreType.DMA,
            pltpu.SemaphoreType.DMA,
            pltpu.SemaphoreType.DMA,
            pltpu.SemaphoreType.DMA,
            pltpu.SemaphoreType.DMA,
        ],
    )
    return fn(src2d, dst2d, stab, dtab)


def _stage_c_body(accp_ref, out_ref):
    a = accp_ref[0] + accp_ref[1]
    colx = lax.broadcasted_iota(jnp.int32, (_N, _H), 1)
    den = jnp.sum(jnp.where(colx == 4, a, 0.0), axis=1, keepdims=True)
    x = a / (den + 1e-16)
    valid = (colx >= 1) & (colx <= 3)
    z = jnp.where(valid, x, -1e30)
    m = jnp.max(z, axis=1, keepdims=True)
    ez = jnp.where(valid, jnp.exp(z - m), 0.0)
    out_ref[...] = ez / jnp.sum(ez, axis=1, keepdims=True)


def _stage_c(accp):
    return pl.pallas_call(
        _stage_c_body,
        out_shape=jax.ShapeDtypeStruct((_N, _H), f32),
    )(accp)


@jax.jit
def kernel(node_embeddings, edge_index, W1, a1_src, a1_dst, W2, a2_src, a2_dst):
    w1r = W1.reshape(_D, _HF)
    hf_ids = jnp.arange(_HF, dtype=jnp.int32)
    asrc = jnp.zeros((_HF, _H), f32).at[hf_ids, hf_ids // _F].set(
        a1_src.reshape(-1))
    adst = jnp.zeros((_HF, _H), f32).at[hf_ids, hf_ids // _F].set(
        a1_dst.reshape(-1))
    m16 = jnp.arange(16, dtype=jnp.int32)
    cols_a = jnp.concatenate([m16, 32 + m16])
    wlo = w1r[:, cols_a]
    whi = w1r[:, cols_a + 16]
    wes = w1r @ asrc
    wed = w1r @ adst
    r8 = jnp.zeros((_H, _HF), f32).at[hf_ids // _F, hf_ids].set(1.0)
    w2r = W2.reshape(_HF, _C)
    wsrc = jnp.zeros((_HF, _H), f32)
    wsrc = wsrc.at[:, 0].set(w2r @ a2_src.reshape(_C))
    wsrc = wsrc.at[:, 1:4].set(w2r)
    wdst = jnp.zeros((_HF, _H), f32).at[:, 0].set(w2r @ a2_dst.reshape(_C))

    src2d = edge_index[0].reshape(_E // _MB, _MB)
    dst2d = edge_index[1].reshape(_E // _MB, _MB)

    hw, es, ed = _stage_a(node_embeddings, wlo, whi, wes, wed)
    msgp, denp = _sc1(src2d, dst2d, hw, es, ed)
    stab, dtab = _stage_b(msgp, denp, r8, wsrc, wdst)
    accp = _sc2(src2d, dst2d, stab, dtab)
    out8 = _stage_c(accp)
    return out8[:, 1:4]
